# baseline (speedup 1.0000x reference)
import functools

import jax
import jax.numpy as jnp
from jax import lax
from jax.experimental import pallas as pl
from jax.experimental.pallas import tpu as pltpu
from jax.experimental.pallas import tpu_sc as plsc

_NC, _NS = 2, 16
_NW = _NC * _NS
_D = 64
_SEG = 128
_KSEG = 8
_CHUNK = _SEG * _KSEG


def _gather_body(n_chunks, ids_hbm, table_hbm, out_hbm, idx_v, rows_v, gsem):
    wid = lax.axis_index("s") * _NC + lax.axis_index("c")

    def body(c, carry):
        g = wid * n_chunks + c
        pltpu.sync_copy(ids_hbm.at[g], idx_v)
        cps = [
            pltpu.async_copy(
                table_hbm.at[idx_v.at[j]],
                rows_v.at[pl.ds(j * _SEG, _SEG)],
                gsem,
            )
            for j in range(_KSEG)
        ]
        for cp in cps:
            cp.wait()
        pltpu.sync_copy(rows_v, out_hbm.at[g])
        return carry

    lax.fori_loop(0, n_chunks, body, 0)


def _sc_gather(ids_r, table, n_chunks):
    kfn = functools.partial(
        pl.kernel,
        mesh=plsc.VectorSubcoreMesh(core_axis_name="c", subcore_axis_name="s"),
        out_type=jax.ShapeDtypeStruct((_NW * n_chunks, _CHUNK, _D), jnp.float32),
        scratch_types=[
            pltpu.VMEM((_KSEG, _SEG), jnp.int32),
            pltpu.VMEM((_CHUNK, _D), jnp.float32),
            pltpu.SemaphoreType.DMA,
        ],
        compiler_params=pltpu.CompilerParams(use_tc_tiling_on_sc=False),
    )(functools.partial(_gather_body, n_chunks))
    return kfn(ids_r, table)


def _mask_body(ids_ref, mask_ref):
    mask_ref[...] = ids_ref[...] != 0


def _tc_mask(ids):
    return pl.pallas_call(
        _mask_body,
        out_shape=jax.ShapeDtypeStruct(ids.shape, jnp.bool_),
    )(ids)


def kernel(ids, table):
    b, t = ids.shape
    total = b * t
    n_chunks = total // (_NW * _CHUNK)
    ids_r = ids.reshape(_NW * n_chunks, _KSEG, _SEG)
    tokens = _sc_gather(ids_r, table, n_chunks).reshape(b, t, _D)
    mask = _tc_mask(ids)
    return tokens, mask


# speedup vs baseline: 1.0152x; 1.0152x over previous
import functools

import jax
import jax.numpy as jnp
from jax import lax
from jax.experimental import pallas as pl
from jax.experimental.pallas import tpu as pltpu
from jax.experimental.pallas import tpu_sc as plsc

_NC, _NS = 2, 16
_NW = _NC * _NS
_D = 64
_SEG = 128
_RING = 8


def _gather_body(n_streams, ids_hbm, table_hbm, out_hbm, idx_v, rows_v, *sems):
    gsem = sems[:_RING]
    osem = sems[_RING:]
    wid = lax.axis_index("s") * _NC + lax.axis_index("c")
    n_rounds = n_streams // _RING

    def fire_gather(b, s):
        return pltpu.async_copy(table_hbm.at[idx_v.at[s]], rows_v.at[b], gsem[b])

    def wait_gather(b, s):
        pltpu.make_async_copy(table_hbm.at[idx_v.at[s]], rows_v.at[b], gsem[b]).wait()

    def fire_wb(b, s):
        return pltpu.async_copy(rows_v.at[b], out_hbm.at[wid, s], osem[b])

    def wait_wb(b, s):
        pltpu.make_async_copy(rows_v.at[b], out_hbm.at[wid, s], osem[b]).wait()

    pltpu.sync_copy(ids_hbm.at[wid], idx_v)

    for b in range(_RING):
        fire_gather(b, b)

    def round_body(r, carry):
        for b in range(_RING):
            s = r * _RING + b
            wait_gather(b, s)
            fire_wb(b, s)
        for b in range(_RING):
            s = r * _RING + b
            wait_wb(b, s)
            fire_gather(b, s + _RING)
        return carry

    lax.fori_loop(0, n_rounds - 1, round_body, 0)

    r = n_rounds - 1
    for b in range(_RING):
        s = r * _RING + b
        wait_gather(b, s)
        fire_wb(b, s)
    for b in range(_RING):
        wait_wb(b, r * _RING + b)


def _sc_gather(ids_r, table, n_streams):
    kfn = functools.partial(
        pl.kernel,
        mesh=plsc.VectorSubcoreMesh(core_axis_name="c", subcore_axis_name="s"),
        out_type=jax.ShapeDtypeStruct((_NW, n_streams, _SEG, _D), jnp.float32),
        scratch_types=[
            pltpu.VMEM((n_streams, _SEG), jnp.int32),
            pltpu.VMEM((_RING, _SEG, _D), jnp.float32),
        ] + [pltpu.SemaphoreType.DMA] * (2 * _RING),
        compiler_params=pltpu.CompilerParams(use_tc_tiling_on_sc=False),
    )(functools.partial(_gather_body, n_streams))
    return kfn(ids_r, table)


def _mask_body(ids_ref, mask_ref):
    mask_ref[...] = ids_ref[...] != 0


def _tc_mask(ids):
    return pl.pallas_call(
        _mask_body,
        out_shape=jax.ShapeDtypeStruct(ids.shape, jnp.bool_),
    )(ids)


def kernel(ids, table):
    b, t = ids.shape
    total = b * t
    n_streams = total // (_NW * _SEG)
    assert n_streams * _NW * _SEG == total and n_streams % _RING == 0
    ids_r = ids.reshape(_NW, n_streams, _SEG)
    tokens = _sc_gather(ids_r, table, n_streams).reshape(b, t, _D)
    mask = _tc_mask(ids)
    return tokens, mask
